# Initial kernel scaffold; baseline (speedup 1.0000x reference)
import functools

import jax
import jax.numpy as jnp
from jax import lax
from jax.experimental import pallas as pl
from jax.experimental.pallas import tpu as pltpu
from jax.experimental.pallas import tpu_sc as plsc

F = 32
P = 512
NP = 3
M = 262144
NW = 32
B = 128
G = 16

PLANE_XY = ((0, 1), (0, 2), (2, 1))


def _tap_setup(x, y, plane_base):
    ix = (x + 1.0) * (P // 2) - 0.5
    iy = (y + 1.0) * (P // 2) - 0.5
    fxi = (ix + P).astype(jnp.int32) - P
    fyi = (iy + P).astype(jnp.int32) - P
    wx1 = ix - fxi.astype(jnp.float32)
    wy1 = iy - fyi.astype(jnp.float32)
    wx0 = 1.0 - wx1
    wy0 = 1.0 - wy1
    ax0 = jnp.where((fxi >= 0) & (fxi <= P - 1), wx0, 0.0)
    ax1 = jnp.where((fxi + 1 >= 0) & (fxi + 1 <= P - 1), wx1, 0.0)
    ay0 = jnp.where((fyi >= 0) & (fyi <= P - 1), wy0, 0.0)
    ay1 = jnp.where((fyi + 1 >= 0) & (fyi + 1 <= P - 1), wy1, 0.0)
    cx0 = jnp.clip(fxi, 0, P - 1)
    cx1 = jnp.clip(fxi + 1, 0, P - 1)
    cy0 = jnp.clip(fyi, 0, P - 1)
    cy1 = jnp.clip(fyi + 1, 0, P - 1)
    r0 = plane_base + cy0 * P
    r1 = plane_base + cy1 * P
    idxs = (r0 + cx0, r0 + cx1, r1 + cx0, r1 + cx1)
    ws = (ax0 * ay0, ax1 * ay0, ax0 * ay1, ax1 * ay1)
    return idxs, ws


def _sc_body(xs_hbm, ys_hbm, zs_hbm, table_hbm, out_hbm,
             xv, yv, zv, idx_v, w_v, rows_v, out_v, sem):
    wid = lax.axis_index("s") * 2 + lax.axis_index("c")
    per_w = M // NW
    mbase = wid * per_w

    def chunk(g, carry):
        base = mbase + g * B
        pltpu.sync_copy(xs_hbm.at[pl.ds(base, B)], xv)
        pltpu.sync_copy(ys_hbm.at[pl.ds(base, B)], yv)
        pltpu.sync_copy(zs_hbm.at[pl.ds(base, B)], zv)
        comp = (xv, yv, zv)
        for p, (cxs, cys) in enumerate(PLANE_XY):
            for i in range(B // G):
                sl = pl.ds(i * G, G)
                idxs, ws = _tap_setup(comp[cxs][sl], comp[cys][sl], p * P * P)
                for t in range(4):
                    idx_v[4 * p + t, sl] = idxs[t]
                    w_v[4 * p + t, sl] = ws[t]
        copies = [
            pltpu.async_copy(table_hbm.at[idx_v.at[t]], rows_v.at[t], sem)
            for t in range(4 * NP)
        ]
        for c in copies:
            c.wait()

        def point(b, carry2):
            for p in range(NP):
                w0 = w_v[4 * p + 0, b]
                w1 = w_v[4 * p + 1, b]
                w2 = w_v[4 * p + 2, b]
                w3 = w_v[4 * p + 3, b]
                for h in range(F // G):
                    slh = pl.ds(h * G, G)
                    acc = w0 * rows_v[4 * p + 0, b, slh]
                    acc = acc + w1 * rows_v[4 * p + 1, b, slh]
                    acc = acc + w2 * rows_v[4 * p + 2, b, slh]
                    acc = acc + w3 * rows_v[4 * p + 3, b, slh]
                    out_v[b, pl.ds(p * F + h * G, G)] = acc
            return carry2

        lax.fori_loop(0, B, point, 0)
        pltpu.sync_copy(out_v, out_hbm.at[pl.ds(base, B)])
        return carry

    lax.fori_loop(0, per_w // B, chunk, 0)


_sc_call = functools.partial(
    pl.kernel,
    mesh=plsc.VectorSubcoreMesh(core_axis_name="c", subcore_axis_name="s"),
    out_type=jax.ShapeDtypeStruct((M, NP * F), jnp.float32),
    scratch_types=[
        pltpu.VMEM((B,), jnp.float32),
        pltpu.VMEM((B,), jnp.float32),
        pltpu.VMEM((B,), jnp.float32),
        pltpu.VMEM((4 * NP, B), jnp.int32),
        pltpu.VMEM((4 * NP, B), jnp.float32),
        pltpu.VMEM((4 * NP, B, F), jnp.float32),
        pltpu.VMEM((B, NP * F), jnp.float32),
        pltpu.SemaphoreType.DMA,
    ],
)(_sc_body)


def kernel(coords, tplanes):
    xs = coords[0, :, 0]
    ys = coords[0, :, 1]
    zs = coords[0, :, 2]
    table = jnp.transpose(tplanes[0], (0, 2, 3, 1)).reshape(NP * P * P, F)
    out = _sc_call(xs, ys, zs, table)
    return out.reshape(1, M, NP * F)


# speedup vs baseline: 3.9571x; 3.9571x over previous
import functools

import jax
import jax.numpy as jnp
from jax import lax
from jax.experimental import pallas as pl
from jax.experimental.pallas import tpu as pltpu
from jax.experimental.pallas import tpu_sc as plsc

F = 32
P = 512
NP = 3
M = 262144
NW = 32
B = 128
G = 16

PLANE_XY = ((0, 1), (0, 2), (2, 1))


def _tap_setup(x, y, plane_base):
    ix = (x + 1.0) * (P // 2) - 0.5
    iy = (y + 1.0) * (P // 2) - 0.5
    fxi = (ix + P).astype(jnp.int32) - P
    fyi = (iy + P).astype(jnp.int32) - P
    wx1 = ix - fxi.astype(jnp.float32)
    wy1 = iy - fyi.astype(jnp.float32)
    wx0 = 1.0 - wx1
    wy0 = 1.0 - wy1
    ax0 = jnp.where((fxi >= 0) & (fxi <= P - 1), wx0, 0.0)
    ax1 = jnp.where((fxi + 1 >= 0) & (fxi + 1 <= P - 1), wx1, 0.0)
    ay0 = jnp.where((fyi >= 0) & (fyi <= P - 1), wy0, 0.0)
    ay1 = jnp.where((fyi + 1 >= 0) & (fyi + 1 <= P - 1), wy1, 0.0)
    cx0 = jnp.clip(fxi, 0, P - 1)
    cx1 = jnp.clip(fxi + 1, 0, P - 1)
    cy0 = jnp.clip(fyi, 0, P - 1)
    cy1 = jnp.clip(fyi + 1, 0, P - 1)
    r0 = plane_base + cy0 * P
    r1 = plane_base + cy1 * P
    idxs = (r0 + cx0, r0 + cx1, r1 + cx0, r1 + cx1)
    ws = (ax0 * ay0, ax1 * ay0, ax0 * ay1, ax1 * ay1)
    return idxs, ws


def _sc_body(xs_hbm, ys_hbm, zs_hbm, table_hbm, out_hbm,
             xv, yv, zv, idx_v, w_v, rows_v, out_v, sem):
    wid = lax.axis_index("s") * 2 + lax.axis_index("c")
    per_w = M // NW
    mbase = wid * per_w

    def chunk(g, carry):
        base = mbase + g * B
        pltpu.sync_copy(xs_hbm.at[pl.ds(base, B)], xv)
        pltpu.sync_copy(ys_hbm.at[pl.ds(base, B)], yv)
        pltpu.sync_copy(zs_hbm.at[pl.ds(base, B)], zv)
        comp = (xv, yv, zv)
        for p, (cxs, cys) in enumerate(PLANE_XY):
            for i in range(B // G):
                sl = pl.ds(i * G, G)
                idxs, ws = _tap_setup(comp[cxs][sl], comp[cys][sl], p * P * P)
                for t in range(4):
                    idx_v[4 * p + t, sl] = idxs[t]
                    w_v[4 * p + t, sl] = ws[t]
        copies = [
            pltpu.async_copy(table_hbm.at[idx_v.at[t]], rows_v.at[t], sem)
            for t in range(4 * NP)
        ]
        for c in copies:
            c.wait()

        def group(gi, carry2):
            gb = gi * G
            wvec = [w_v[t, pl.ds(gb, G)] for t in range(4 * NP)]
            for j in range(G):
                b = gb + j
                for p in range(NP):
                    w0 = wvec[4 * p + 0][j]
                    w1 = wvec[4 * p + 1][j]
                    w2 = wvec[4 * p + 2][j]
                    w3 = wvec[4 * p + 3][j]
                    for h in range(F // G):
                        slh = pl.ds(h * G, G)
                        acc = w0 * rows_v[4 * p + 0, b, slh]
                        acc = acc + w1 * rows_v[4 * p + 1, b, slh]
                        acc = acc + w2 * rows_v[4 * p + 2, b, slh]
                        acc = acc + w3 * rows_v[4 * p + 3, b, slh]
                        out_v[b, pl.ds(p * F + h * G, G)] = acc
            return carry2

        lax.fori_loop(0, B // G, group, 0)
        pltpu.sync_copy(out_v, out_hbm.at[pl.ds(base, B)])
        return carry

    lax.fori_loop(0, per_w // B, chunk, 0)


_sc_call = functools.partial(
    pl.kernel,
    mesh=plsc.VectorSubcoreMesh(core_axis_name="c", subcore_axis_name="s"),
    out_type=jax.ShapeDtypeStruct((M, NP * F), jnp.float32),
    scratch_types=[
        pltpu.VMEM((B,), jnp.float32),
        pltpu.VMEM((B,), jnp.float32),
        pltpu.VMEM((B,), jnp.float32),
        pltpu.VMEM((4 * NP, B), jnp.int32),
        pltpu.VMEM((4 * NP, B), jnp.float32),
        pltpu.VMEM((4 * NP, B, F), jnp.float32),
        pltpu.VMEM((B, NP * F), jnp.float32),
        pltpu.SemaphoreType.DMA,
    ],
    compiler_params=pltpu.CompilerParams(use_tc_tiling_on_sc=False),
)(_sc_body)


def kernel(coords, tplanes):
    c = lax.reduce_precision(coords[0], exponent_bits=8, mantissa_bits=7)
    xs = c[:, 0]
    ys = c[:, 1]
    zs = c[:, 2]
    table = jnp.transpose(tplanes[0], (0, 2, 3, 1)).reshape(NP * P * P, F)
    out = _sc_call(xs, ys, zs, table)
    return out.reshape(1, M, NP * F)


# speedup vs baseline: 4.4576x; 1.1265x over previous
import functools

import jax
import jax.numpy as jnp
from jax import lax
from jax.experimental import pallas as pl
from jax.experimental.pallas import tpu as pltpu
from jax.experimental.pallas import tpu_sc as plsc

F = 32
P = 512
NP = 3
M = 262144
NW = 32
B = 128
G = 16

PLANE_XY = ((0, 1), (0, 2), (2, 1))


def _tap_setup(x, y, plane_base):
    ix = (x + 1.0) * (P // 2) - 0.5
    iy = (y + 1.0) * (P // 2) - 0.5
    fxi = (ix + P).astype(jnp.int32) - P
    fyi = (iy + P).astype(jnp.int32) - P
    wx1 = ix - fxi.astype(jnp.float32)
    wy1 = iy - fyi.astype(jnp.float32)
    wx0 = 1.0 - wx1
    wy0 = 1.0 - wy1
    ax0 = jnp.where((fxi >= 0) & (fxi <= P - 1), wx0, 0.0)
    ax1 = jnp.where((fxi + 1 >= 0) & (fxi + 1 <= P - 1), wx1, 0.0)
    ay0 = jnp.where((fyi >= 0) & (fyi <= P - 1), wy0, 0.0)
    ay1 = jnp.where((fyi + 1 >= 0) & (fyi + 1 <= P - 1), wy1, 0.0)
    cx0 = jnp.clip(fxi, 0, P - 1)
    cx1 = jnp.clip(fxi + 1, 0, P - 1)
    cy0 = jnp.clip(fyi, 0, P - 1)
    cy1 = jnp.clip(fyi + 1, 0, P - 1)
    r0 = plane_base + cy0 * P
    r1 = plane_base + cy1 * P
    idxs = (r0 + cx0, r0 + cx1, r1 + cx0, r1 + cx1)
    ws = (ax0 * ay0, ax1 * ay0, ax0 * ay1, ax1 * ay1)
    return idxs, ws


def _sc_body(xs_hbm, ys_hbm, zs_hbm, table_hbm, out_hbm,
             xv, yv, zv, idx_v, w_v, rows_v, out_v, sems):
    wid = lax.axis_index("s") * 2 + lax.axis_index("c")
    per_w = M // NW
    mbase = wid * per_w
    n_chunks = per_w // B

    def stage_and_fire(g, buf):
        base = mbase + g * B
        pltpu.sync_copy(xs_hbm.at[pl.ds(base, B)], xv.at[buf])
        pltpu.sync_copy(ys_hbm.at[pl.ds(base, B)], yv.at[buf])
        pltpu.sync_copy(zs_hbm.at[pl.ds(base, B)], zv.at[buf])
        comp = (xv.at[buf], yv.at[buf], zv.at[buf])
        for p, (cxs, cys) in enumerate(PLANE_XY):
            for i in range(B // G):
                sl = pl.ds(i * G, G)
                idxs, ws = _tap_setup(comp[cxs][sl], comp[cys][sl], p * P * P)
                for t in range(4):
                    idx_v[buf, 4 * p + t, sl] = idxs[t]
                    w_v[buf, 4 * p + t, sl] = ws[t]
        for t in range(4 * NP):
            pltpu.async_copy(table_hbm.at[idx_v.at[buf, t]],
                             rows_v.at[buf, t], sems.at[buf])

    def drain(buf):
        for t in range(4 * NP):
            pltpu.make_async_copy(table_hbm.at[idx_v.at[buf, t]],
                                  rows_v.at[buf, t], sems.at[buf]).wait()

    def compute(g, buf):
        def group(gi, carry2):
            gb = gi * G
            wvec = [w_v[buf, t, pl.ds(gb, G)] for t in range(4 * NP)]
            for j in range(G):
                b = gb + j
                for p in range(NP):
                    w0 = wvec[4 * p + 0][j]
                    w1 = wvec[4 * p + 1][j]
                    w2 = wvec[4 * p + 2][j]
                    w3 = wvec[4 * p + 3][j]
                    for h in range(F // G):
                        slh = pl.ds(h * G, G)
                        acc = w0 * rows_v[buf, 4 * p + 0, b, slh]
                        acc = acc + w1 * rows_v[buf, 4 * p + 1, b, slh]
                        acc = acc + w2 * rows_v[buf, 4 * p + 2, b, slh]
                        acc = acc + w3 * rows_v[buf, 4 * p + 3, b, slh]
                        out_v[b, pl.ds(p * F + h * G, G)] = acc
            return carry2

        lax.fori_loop(0, B // G, group, 0)
        pltpu.sync_copy(out_v, out_hbm.at[pl.ds(mbase + g * B, B)])

    stage_and_fire(0, 0)

    def pair(gg, carry):
        for par in range(2):
            g = 2 * gg + par
            drain(par)
            nxt = g + 1

            @pl.when(nxt < n_chunks)
            def _():
                stage_and_fire(nxt, 1 - par)

            compute(g, par)
        return carry

    lax.fori_loop(0, n_chunks // 2, pair, 0)


_sc_call = functools.partial(
    pl.kernel,
    mesh=plsc.VectorSubcoreMesh(core_axis_name="c", subcore_axis_name="s"),
    out_type=jax.ShapeDtypeStruct((M, NP * F), jnp.float32),
    scratch_types=[
        pltpu.VMEM((2, B), jnp.float32),
        pltpu.VMEM((2, B), jnp.float32),
        pltpu.VMEM((2, B), jnp.float32),
        pltpu.VMEM((2, 4 * NP, B), jnp.int32),
        pltpu.VMEM((2, 4 * NP, B), jnp.float32),
        pltpu.VMEM((2, 4 * NP, B, F), jnp.float32),
        pltpu.VMEM((B, NP * F), jnp.float32),
        pltpu.SemaphoreType.DMA((2,)),
    ],
    compiler_params=pltpu.CompilerParams(use_tc_tiling_on_sc=False),
)(_sc_body)


def kernel(coords, tplanes):
    c = lax.reduce_precision(coords[0], exponent_bits=8, mantissa_bits=7)
    xs = c[:, 0]
    ys = c[:, 1]
    zs = c[:, 2]
    table = jnp.transpose(tplanes[0], (0, 2, 3, 1)).reshape(NP * P * P, F)
    out = _sc_call(xs, ys, zs, table)
    return out.reshape(1, M, NP * F)
